# baseline (speedup 1.0000x reference)
import functools

import jax
import jax.numpy as jnp
from jax import lax
from jax.experimental import pallas as pl
from jax.experimental.pallas import tpu as pltpu
from jax.experimental.pallas import tpu_sc as plsc

_DICT = 100000
_NW = 32
_BLK = 480
_G = 16
_NL = 16
_MW = 128

_params = pltpu.CompilerParams(needs_layout_passes=False)


def _sload(ref, *idx):
    v = plsc.load_gather(ref, [jnp.full((_NL,), i, jnp.int32) for i in idx])
    return v[0]


def _row_vload(ref, row, col0):
    lanes = lax.iota(jnp.int32, _NL)
    return plsc.load_gather(
        ref, [jnp.full((_NL,), row, jnp.int32), col0 + lanes])


def _row_vstore(ref, row, col0, val):
    lanes = lax.iota(jnp.int32, _NL)
    plsc.store_scatter(
        ref, [jnp.full((_NL,), row, jnp.int32), col0 + lanes], val)


def _build_table(base_weight, extract_index, scatter_index, msk0):
    V, D = base_weight.shape
    T = extract_index.shape[0]
    nvec = D // _NL
    chunk = ((T // _NW) + _G - 1) // _G * _G
    ngrp = chunk // _G
    mesh = plsc.VectorSubcoreMesh(core_axis_name="c", subcore_axis_name="s")

    @functools.partial(
        pl.kernel,
        mesh=mesh,
        compiler_params=_params,
        out_type=(
            jax.ShapeDtypeStruct((_DICT, D), jnp.float32),
            jax.ShapeDtypeStruct((2 * _NW, D), jnp.float32),
            jax.ShapeDtypeStruct((2 * _NW, _NL), jnp.int32),
        ),
        scratch_types=[
            pltpu.VMEM((2 * _BLK,), jnp.int32),
            pltpu.VMEM((2 * _BLK,), jnp.int32),
            pltpu.VMEM((2, _G, D), jnp.float32),
            pltpu.VMEM((4, D), jnp.float32),
            pltpu.VMEM((D,), jnp.float32),
            pltpu.VMEM((_MW,), jnp.float32),
            pltpu.VMEM((_NL,), jnp.int32),
            pltpu.SemaphoreType.DMA((2,)),
            pltpu.SemaphoreType.DMA((4,)),
        ],
    )
    def build(base_hbm, eidx_hbm, sidx_hbm, msk_hbm,
              table_hbm, psum_hbm, pmeta_hbm,
              sidx_v, eidx_v, rows_v, stage_v, acc_v, ones_v, meta_v,
              sem_g, sem_s):
        wid = lax.axis_index("s") * 2 + lax.axis_index("c")
        c0 = pl.multiple_of(wid * chunk, _G)
        c1 = jnp.minimum(c0 + chunk, T)
        lanes = lax.iota(jnp.int32, _NL)

        for k in range(_MW // _NL):
            ones_v[pl.ds(k * _NL, _NL)] = jnp.ones((_NL,), jnp.float32)

        a0 = pl.multiple_of((c0 // _BLK) * _BLK, _BLK)
        bs0 = (c0 // _BLK) % 2
        boffs0 = pl.multiple_of(bs0 * _BLK, 8)
        pltpu.sync_copy(sidx_hbm.at[pl.ds(a0, _BLK)],
                        sidx_v.at[pl.ds(boffs0, _BLK)])
        pltpu.sync_copy(eidx_hbm.at[pl.ds(a0, _BLK)],
                        eidx_v.at[pl.ds(boffs0, _BLK)])
        cur0 = _sload(sidx_v, c0 - a0 + bs0 * _BLK)

        bo0 = pl.multiple_of(c0 - a0 + bs0 * _BLK, _G)
        pltpu.async_copy(base_hbm.at[eidx_v.at[pl.ds(bo0, _G)]],
                         rows_v.at[0], sem_g.at[0])

        def write_partial(slot, seg, cnt):
            pltpu.sync_copy(acc_v, psum_hbm.at[slot])
            meta = jnp.where(lanes == 0, seg,
                             jnp.where(lanes == 1, cnt, jnp.int32(0)))
            meta_v[pl.ds(0, _NL)] = meta
            pltpu.sync_copy(meta_v, pmeta_hbm.at[slot])

        def body(g, st):
            cur, cnt, first, ncl = st
            i0 = c0 + g * _G
            live = i0 < c1
            i0n = i0 + _G
            liven = i0n < c1
            p = g % 2
            pn = 1 - p

            @pl.when(liven & (i0n % _BLK == 0))
            def _():
                ian = pl.multiple_of(i0n, _BLK)
                bsn = pl.multiple_of(((i0n // _BLK) % 2) * _BLK, 8)
                pltpu.sync_copy(sidx_hbm.at[pl.ds(ian, _BLK)],
                                sidx_v.at[pl.ds(bsn, _BLK)])
                pltpu.sync_copy(eidx_hbm.at[pl.ds(ian, _BLK)],
                                eidx_v.at[pl.ds(bsn, _BLK)])

            @pl.when(live)
            def _():
                pltpu.make_async_copy(base_hbm.at[pl.ds(0, _G)],
                                      rows_v.at[p], sem_g.at[p]).wait()

            @pl.when(liven)
            def _():
                bon = pl.multiple_of(
                    i0n - (i0n // _BLK) * _BLK
                    + ((i0n // _BLK) % 2) * _BLK, _G)
                pltpu.async_copy(base_hbm.at[eidx_v.at[pl.ds(bon, _G)]],
                                 rows_v.at[pn], sem_g.at[pn])

            bo = i0 - (i0 // _BLK) * _BLK + ((i0 // _BLK) % 2) * _BLK
            sv_vec = plsc.load_gather(sidx_v, [jnp.full((_NL,), bo,
                                                        jnp.int32) + lanes])

            for r in range(_G):
                sv = sv_vec[r]
                is_b = live & (sv != cur)
                close_int = is_b & (first == 0)
                close_first = is_b & (first == 1)

                @pl.when(close_int)
                def _(cur=cur, cnt=cnt, ncl=ncl):
                    slot = ncl % 4

                    @pl.when(ncl >= 4)
                    def _():
                        pltpu.make_async_copy(
                            table_hbm.at[0], stage_v.at[slot],
                            sem_s.at[slot]).wait()
                        pltpu.make_async_copy(
                            msk_hbm.at[0], ones_v, sem_s.at[slot]).wait()

                    den = jnp.full((_NL,), cnt.astype(jnp.float32))
                    inv = jnp.ones((_NL,), jnp.float32) / den
                    for k in range(nvec):
                        sl = pl.ds(k * _NL, _NL)
                        stage_v[slot, sl] = acc_v[sl] * inv
                    pltpu.async_copy(stage_v.at[slot], table_hbm.at[cur],
                                     sem_s.at[slot])
                    pltpu.async_copy(ones_v, msk_hbm.at[cur],
                                     sem_s.at[slot])

                @pl.when(close_first)
                def _(cur=cur, cnt=cnt):
                    write_partial(2 * wid, cur, cnt)

                @pl.when(live)
                def _(is_b=is_b):
                    for k in range(nvec):
                        sl = pl.ds(k * _NL, _NL)
                        row_k = rows_v[p, r, sl]
                        acc_v[sl] = jnp.where(is_b, row_k,
                                              acc_v[sl] + row_k)

                cnt = jnp.where(is_b, jnp.int32(1),
                                jnp.where(live, cnt + 1, cnt))
                cur = jnp.where(is_b, sv, cur)
                ncl = jnp.where(close_int, ncl + 1, ncl)
                first = jnp.where(is_b, jnp.int32(0), first)

            return (cur, cnt, first, ncl)

        cur_f, cnt_f, first_f, ncl_f = lax.fori_loop(
            0, ngrp, body, (cur0, jnp.int32(0), jnp.int32(1), jnp.int32(0)))

        @pl.when(first_f == 1)
        def _():
            write_partial(2 * wid, cur_f, cnt_f)
            meta_v[pl.ds(0, _NL)] = jnp.where(
                lanes == 0, jnp.int32(-1), jnp.int32(0))
            pltpu.sync_copy(meta_v, pmeta_hbm.at[2 * wid + 1])

        @pl.when(first_f == 0)
        def _():
            write_partial(2 * wid + 1, cur_f, cnt_f)

        for k in range(4):
            @pl.when(ncl_f > k)
            def _(k=k):
                pltpu.make_async_copy(
                    table_hbm.at[0], stage_v.at[k], sem_s.at[k]).wait()
                pltpu.make_async_copy(
                    msk_hbm.at[0], ones_v, sem_s.at[k]).wait()

    return build(base_weight, extract_index, scatter_index, msk0)


def _combine_gather(table, msk, psum, pmeta, flat_tokens):
    N = flat_tokens.shape[0]
    D = table.shape[1]
    nvec = D // _NL
    per_w = N // _NW
    blk = 64
    nparts = pmeta.shape[0]
    mesh = plsc.VectorSubcoreMesh(core_axis_name="c", subcore_axis_name="s")

    @functools.partial(
        pl.kernel,
        mesh=mesh,
        compiler_params=_params,
        out_type=jax.ShapeDtypeStruct((N, D), jnp.float32),
        scratch_types=[
            pltpu.VMEM((2 * _NW, D), jnp.float32),
            pltpu.VMEM((2 * _NW, _NL), jnp.int32),
            pltpu.VMEM((D,), jnp.float32),
            pltpu.VMEM((D,), jnp.float32),
            pltpu.VMEM((_MW,), jnp.float32),
            pltpu.VMEM((64,), jnp.int32),
            pltpu.VMEM((64, D), jnp.float32),
            pltpu.VMEM((64, _MW), jnp.float32),
            pltpu.SemaphoreType.DMA,
            pltpu.SemaphoreType.DMA,
        ],
    )
    def gat(table_hbm, msk_hbm, psum_hbm, pmeta_hbm, tok_hbm, out_hbm,
            parts_v, pmeta_v, sacc_v, sstage_v, ones_v, idx_v, rows_v,
            mrows_v, sem, sem2):
        sid = lax.axis_index("s")
        wid = sid * 2 + lax.axis_index("c")
        zv = jnp.zeros((_NL,), jnp.float32)

        @pl.when(sid == 0)
        def _():
            for k in range(_MW // _NL):
                ones_v[pl.ds(k * _NL, _NL)] = jnp.ones((_NL,), jnp.float32)
            pltpu.sync_copy(psum_hbm, parts_v)
            pltpu.sync_copy(pmeta_hbm, pmeta_v)

            def close(seg, cnt):
                den = jnp.full((_NL,), cnt.astype(jnp.float32))
                inv = jnp.ones((_NL,), jnp.float32) / den
                for k in range(nvec):
                    sl = pl.ds(k * _NL, _NL)
                    sstage_v[sl] = sacc_v[sl] * inv
                pltpu.sync_copy(sstage_v, table_hbm.at[seg])
                pltpu.sync_copy(ones_v, msk_hbm.at[seg])

            def body(e, st):
                cur, cnt = st
                id_e = _sload(pmeta_v, e, 0)
                cnt_e = _sload(pmeta_v, e, 1)
                valid = cnt_e > 0
                same = valid & (id_e == cur)
                newseg = valid & jnp.logical_not(same)
                @pl.when(newseg & (cur >= 0))
                def _():
                    close(cur, cnt)

                @pl.when(newseg)
                def _():
                    for k in range(nvec):
                        sacc_v[pl.ds(k * _NL, _NL)] = _row_vload(
                            parts_v, e, k * _NL)

                @pl.when(same)
                def _():
                    for k in range(nvec):
                        sl = pl.ds(k * _NL, _NL)
                        sacc_v[sl] = sacc_v[sl] + _row_vload(
                            parts_v, e, k * _NL)

                cur_new = jnp.where(newseg, id_e, cur)
                cnt_new = jnp.where(newseg, cnt_e,
                                    jnp.where(same, cnt + cnt_e, cnt))
                return (cur_new, cnt_new)

            cur_f, cnt_f = lax.fori_loop(
                0, nparts, body, (jnp.int32(-1), jnp.int32(0)))

            @pl.when(cur_f >= 0)
            def _():
                close(cur_f, cnt_f)

        plsc.subcore_barrier()

        base = wid * per_w

        def gbody(b, carry):
            off = pl.multiple_of(base + b * blk, blk)
            pltpu.sync_copy(tok_hbm.at[pl.ds(off, blk)], idx_v)
            cp1 = pltpu.async_copy(table_hbm.at[idx_v], rows_v, sem)
            cp2 = pltpu.async_copy(msk_hbm.at[idx_v], mrows_v, sem2)
            cp1.wait()
            cp2.wait()

            def mbody(j, mcarry):
                m = _sload(mrows_v, j, 0)

                @pl.when(m == 0.0)
                def _():
                    for k in range(nvec):
                        _row_vstore(rows_v, j, k * _NL, zv)
                return mcarry

            lax.fori_loop(0, blk, mbody, jnp.int32(0))
            pltpu.sync_copy(rows_v, out_hbm.at[pl.ds(off, blk)])
            return carry

        lax.fori_loop(0, per_w // blk, gbody, jnp.int32(0))

    return gat(table, msk, psum, pmeta, flat_tokens)


def kernel(base_weight, extract_index, scatter_index, token_index):
    B, L = token_index.shape
    D = base_weight.shape[1]
    z0 = base_weight[0, 0] * 0.0
    msk0 = jnp.full((_DICT, _MW), 0.0, jnp.float32) + z0
    table, psum, pmeta = _build_table(
        base_weight, extract_index, scatter_index, msk0)
    out = _combine_gather(table, msk0, psum, pmeta, token_index.reshape(-1))
    return out.reshape(B, L, D)


# speedup vs baseline: 2.5293x; 2.5293x over previous
import functools

import jax
import jax.numpy as jnp
from jax import lax
from jax.experimental import pallas as pl
from jax.experimental.pallas import tpu as pltpu
from jax.experimental.pallas import tpu_sc as plsc

_DICT = 100000
_NW = 32
_BLK = 480
_G = 16
_NL = 16
_MW = 128

_params = pltpu.CompilerParams(needs_layout_passes=False)


def _sload(ref, *idx):
    v = plsc.load_gather(ref, [jnp.full((_NL,), i, jnp.int32) for i in idx])
    return v[0]


def _row_vload(ref, row, col0):
    lanes = lax.iota(jnp.int32, _NL)
    return plsc.load_gather(
        ref, [jnp.full((_NL,), row, jnp.int32), col0 + lanes])


def _row_vstore(ref, row, col0, val):
    lanes = lax.iota(jnp.int32, _NL)
    plsc.store_scatter(
        ref, [jnp.full((_NL,), row, jnp.int32), col0 + lanes], val)


def _build_table(base_weight, extract_index, scatter_index, msk0):
    V, D = base_weight.shape
    T = extract_index.shape[0]
    nvec = D // _NL
    chunk = ((T // _NW) + _G - 1) // _G * _G
    ngrp = chunk // _G
    mesh = plsc.VectorSubcoreMesh(core_axis_name="c", subcore_axis_name="s")

    @functools.partial(
        pl.kernel,
        mesh=mesh,
        compiler_params=_params,
        out_type=(
            jax.ShapeDtypeStruct((_DICT, D), jnp.float32),
            jax.ShapeDtypeStruct((2 * _NW, D), jnp.float32),
            jax.ShapeDtypeStruct((2 * _NW, _NL), jnp.int32),
        ),
        scratch_types=[
            pltpu.VMEM((2 * _BLK,), jnp.int32),
            pltpu.VMEM((2 * _BLK,), jnp.int32),
            pltpu.VMEM((2, _G, D), jnp.float32),
            pltpu.VMEM((4, D), jnp.float32),
            pltpu.VMEM((D,), jnp.float32),
            pltpu.VMEM((_MW,), jnp.float32),
            pltpu.VMEM((_NL,), jnp.int32),
            pltpu.SemaphoreType.DMA((2,)),
            pltpu.SemaphoreType.DMA((4,)),
        ],
    )
    def build(base_hbm, eidx_hbm, sidx_hbm, msk_hbm,
              table_hbm, psum_hbm, pmeta_hbm,
              sidx_v, eidx_v, rows_v, stage_v, acc_v, ones_v, meta_v,
              sem_g, sem_s):
        wid = lax.axis_index("s") * 2 + lax.axis_index("c")
        c0 = pl.multiple_of(wid * chunk, _G)
        c1 = jnp.minimum(c0 + chunk, T)
        lanes = lax.iota(jnp.int32, _NL)

        for k in range(_MW // _NL):
            ones_v[pl.ds(k * _NL, _NL)] = jnp.ones((_NL,), jnp.float32)

        a0 = pl.multiple_of((c0 // _BLK) * _BLK, _BLK)
        bs0 = (c0 // _BLK) % 2
        boffs0 = pl.multiple_of(bs0 * _BLK, 8)
        pltpu.sync_copy(sidx_hbm.at[pl.ds(a0, _BLK)],
                        sidx_v.at[pl.ds(boffs0, _BLK)])
        pltpu.sync_copy(eidx_hbm.at[pl.ds(a0, _BLK)],
                        eidx_v.at[pl.ds(boffs0, _BLK)])
        cur0 = _sload(sidx_v, c0 - a0 + bs0 * _BLK)

        bo0 = pl.multiple_of(c0 - a0 + bs0 * _BLK, _G)
        pg0 = (c0 // _G) % 2
        pltpu.async_copy(base_hbm.at[eidx_v.at[pl.ds(bo0, _G)]],
                         rows_v.at[pg0], sem_g.at[pg0])

        def write_partial(slot, seg, cnt):
            pltpu.sync_copy(acc_v, psum_hbm.at[slot])
            meta = jnp.where(lanes == 0, seg,
                             jnp.where(lanes == 1, cnt, jnp.int32(0)))
            meta_v[pl.ds(0, _NL)] = meta
            pltpu.sync_copy(meta_v, pmeta_hbm.at[slot])

        def body(j, st):
            cur, cnt, first, ncl = st
            i = c0 + j
            live = i < c1
            ig = i // _G
            pg = ig % 2

            @pl.when((i % _G == 0) & live)
            def _():
                inx = i + _G
                @pl.when((inx % _BLK == 0) & (inx < c1))
                def _():
                    ian = pl.multiple_of(inx, _BLK)
                    bsn = pl.multiple_of(((inx // _BLK) % 2) * _BLK, 8)
                    pltpu.sync_copy(sidx_hbm.at[pl.ds(ian, _BLK)],
                                    sidx_v.at[pl.ds(bsn, _BLK)])
                    pltpu.sync_copy(eidx_hbm.at[pl.ds(ian, _BLK)],
                                    eidx_v.at[pl.ds(bsn, _BLK)])

                pltpu.make_async_copy(base_hbm.at[pl.ds(0, _G)],
                                      rows_v.at[pg], sem_g.at[pg]).wait()

                @pl.when(inx < c1)
                def _():
                    bon = pl.multiple_of(
                        inx - (inx // _BLK) * _BLK
                        + ((inx // _BLK) % 2) * _BLK, _G)
                    pltpu.async_copy(
                        base_hbm.at[eidx_v.at[pl.ds(bon, _G)]],
                        rows_v.at[1 - pg], sem_g.at[1 - pg])

            boff = i - (i // _BLK) * _BLK + ((i // _BLK) % 2) * _BLK
            sv = _sload(sidx_v, boff)
            is_b = live & (sv != cur)
            close_int = is_b & (first == 0)
            close_first = is_b & (first == 1)

            @pl.when(close_int)
            def _():
                slot = ncl % 4

                @pl.when(ncl >= 4)
                def _():
                    pltpu.make_async_copy(
                        table_hbm.at[0], stage_v.at[slot],
                        sem_s.at[slot]).wait()
                    pltpu.make_async_copy(
                        msk_hbm.at[0], ones_v, sem_s.at[slot]).wait()

                den = jnp.full((_NL,), cnt.astype(jnp.float32))
                inv = jnp.ones((_NL,), jnp.float32) / den
                for k in range(nvec):
                    sl = pl.ds(k * _NL, _NL)
                    stage_v[slot, sl] = acc_v[sl] * inv
                pltpu.async_copy(stage_v.at[slot], table_hbm.at[cur],
                                 sem_s.at[slot])
                pltpu.async_copy(ones_v, msk_hbm.at[cur], sem_s.at[slot])

            @pl.when(close_first)
            def _():
                write_partial(2 * wid, cur, cnt)

            goff = i - ig * _G

            @pl.when(live)
            def _():
                for k in range(nvec):
                    sl = pl.ds(k * _NL, _NL)
                    row_k = rows_v[pg, goff, sl]
                    acc_v[sl] = jnp.where(is_b, row_k, acc_v[sl] + row_k)

            cnt_new = jnp.where(is_b, jnp.int32(1),
                                jnp.where(live, cnt + 1, cnt))
            cur_new = jnp.where(is_b, sv, cur)
            ncl_new = jnp.where(close_int, ncl + 1, ncl)
            first_new = jnp.where(is_b, jnp.int32(0), first)
            return (cur_new, cnt_new, first_new, ncl_new)

        cur_f, cnt_f, first_f, ncl_f = lax.fori_loop(
            0, chunk, body, (cur0, jnp.int32(0), jnp.int32(1), jnp.int32(0)))

        @pl.when(first_f == 1)
        def _():
            write_partial(2 * wid, cur_f, cnt_f)
            meta_v[pl.ds(0, _NL)] = jnp.where(
                lanes == 0, jnp.int32(-1), jnp.int32(0))
            pltpu.sync_copy(meta_v, pmeta_hbm.at[2 * wid + 1])

        @pl.when(first_f == 0)
        def _():
            write_partial(2 * wid + 1, cur_f, cnt_f)

        for k in range(4):
            @pl.when(ncl_f > k)
            def _(k=k):
                pltpu.make_async_copy(
                    table_hbm.at[0], stage_v.at[k], sem_s.at[k]).wait()
                pltpu.make_async_copy(
                    msk_hbm.at[0], ones_v, sem_s.at[k]).wait()

    return build(base_weight, extract_index, scatter_index, msk0)


def _combine_gather(table, msk, psum, pmeta, flat_tokens):
    N = flat_tokens.shape[0]
    D = table.shape[1]
    nvec = D // _NL
    per_w = N // _NW
    blk = 64
    nparts = pmeta.shape[0]
    mesh = plsc.VectorSubcoreMesh(core_axis_name="c", subcore_axis_name="s")

    @functools.partial(
        pl.kernel,
        mesh=mesh,
        compiler_params=_params,
        out_type=jax.ShapeDtypeStruct((N, D), jnp.float32),
        scratch_types=[
            pltpu.VMEM((2 * _NW, D), jnp.float32),
            pltpu.VMEM((2 * _NW, _NL), jnp.int32),
            pltpu.VMEM((D,), jnp.float32),
            pltpu.VMEM((D,), jnp.float32),
            pltpu.VMEM((_MW,), jnp.float32),
            pltpu.VMEM((64,), jnp.int32),
            pltpu.VMEM((64, D), jnp.float32),
            pltpu.VMEM((64, _MW), jnp.float32),
            pltpu.SemaphoreType.DMA,
            pltpu.SemaphoreType.DMA,
        ],
    )
    def gat(table_hbm, msk_hbm, psum_hbm, pmeta_hbm, tok_hbm, out_hbm,
            parts_v, pmeta_v, sacc_v, sstage_v, ones_v, idx_v, rows_v,
            mrows_v, sem, sem2):
        sid = lax.axis_index("s")
        wid = sid * 2 + lax.axis_index("c")
        zv = jnp.zeros((_NL,), jnp.float32)

        @pl.when(sid == 0)
        def _():
            for k in range(_MW // _NL):
                ones_v[pl.ds(k * _NL, _NL)] = jnp.ones((_NL,), jnp.float32)
            pltpu.sync_copy(psum_hbm, parts_v)
            pltpu.sync_copy(pmeta_hbm, pmeta_v)

            def close(seg, cnt):
                den = jnp.full((_NL,), cnt.astype(jnp.float32))
                inv = jnp.ones((_NL,), jnp.float32) / den
                for k in range(nvec):
                    sl = pl.ds(k * _NL, _NL)
                    sstage_v[sl] = sacc_v[sl] * inv
                pltpu.sync_copy(sstage_v, table_hbm.at[seg])
                pltpu.sync_copy(ones_v, msk_hbm.at[seg])

            def body(e, st):
                cur, cnt = st
                id_e = _sload(pmeta_v, e, 0)
                cnt_e = _sload(pmeta_v, e, 1)
                valid = cnt_e > 0
                same = valid & (id_e == cur)
                newseg = valid & jnp.logical_not(same)
                @pl.when(newseg & (cur >= 0))
                def _():
                    close(cur, cnt)

                @pl.when(newseg)
                def _():
                    for k in range(nvec):
                        sacc_v[pl.ds(k * _NL, _NL)] = _row_vload(
                            parts_v, e, k * _NL)

                @pl.when(same)
                def _():
                    for k in range(nvec):
                        sl = pl.ds(k * _NL, _NL)
                        sacc_v[sl] = sacc_v[sl] + _row_vload(
                            parts_v, e, k * _NL)

                cur_new = jnp.where(newseg, id_e, cur)
                cnt_new = jnp.where(newseg, cnt_e,
                                    jnp.where(same, cnt + cnt_e, cnt))
                return (cur_new, cnt_new)

            cur_f, cnt_f = lax.fori_loop(
                0, nparts, body, (jnp.int32(-1), jnp.int32(0)))

            @pl.when(cur_f >= 0)
            def _():
                close(cur_f, cnt_f)

        plsc.subcore_barrier()

        base = wid * per_w

        def gbody(b, carry):
            off = pl.multiple_of(base + b * blk, blk)
            pltpu.sync_copy(tok_hbm.at[pl.ds(off, blk)], idx_v)
            cp1 = pltpu.async_copy(table_hbm.at[idx_v], rows_v, sem)
            cp2 = pltpu.async_copy(msk_hbm.at[idx_v], mrows_v, sem2)
            cp1.wait()
            cp2.wait()

            def mbody(j, mcarry):
                m = _sload(mrows_v, j, 0)

                @pl.when(m == 0.0)
                def _():
                    for k in range(nvec):
                        _row_vstore(rows_v, j, k * _NL, zv)
                return mcarry

            lax.fori_loop(0, blk, mbody, jnp.int32(0))
            pltpu.sync_copy(rows_v, out_hbm.at[pl.ds(off, blk)])
            return carry

        lax.fori_loop(0, per_w // blk, gbody, jnp.int32(0))

    return gat(table, msk, psum, pmeta, flat_tokens)


def kernel(base_weight, extract_index, scatter_index, token_index):
    B, L = token_index.shape
    D = base_weight.shape[1]
    z0 = base_weight[0, 0] * 0.0
    msk0 = jnp.full((_DICT, _MW), 0.0, jnp.float32) + z0
    table, psum, pmeta = _build_table(
        base_weight, extract_index, scatter_index, msk0)
    out = _combine_gather(table, msk0, psum, pmeta, token_index.reshape(-1))
    return out.reshape(B, L, D)


# speedup vs baseline: 3.1294x; 1.2372x over previous
import functools

import jax
import jax.numpy as jnp
from jax import lax
from jax.experimental import pallas as pl
from jax.experimental.pallas import tpu as pltpu
from jax.experimental.pallas import tpu_sc as plsc

_DICT = 100000
_NW = 32
_BLK = 480
_G = 16
_NL = 16
_MW = 128

_params = pltpu.CompilerParams(needs_layout_passes=False)


def _sload(ref, *idx):
    v = plsc.load_gather(ref, [jnp.full((_NL,), i, jnp.int32) for i in idx])
    return v[0]


def _row_vload(ref, row, col0):
    lanes = lax.iota(jnp.int32, _NL)
    return plsc.load_gather(
        ref, [jnp.full((_NL,), row, jnp.int32), col0 + lanes])


def _row_vstore(ref, row, col0, val):
    lanes = lax.iota(jnp.int32, _NL)
    plsc.store_scatter(
        ref, [jnp.full((_NL,), row, jnp.int32), col0 + lanes], val)


def _build_table(base_weight, extract_index, scatter_index, msk0):
    V, D = base_weight.shape
    T = extract_index.shape[0]
    nvec = D // _NL
    chunk = ((T // _NW) + _G - 1) // _G * _G
    ngrp = chunk // _G
    mesh = plsc.VectorSubcoreMesh(core_axis_name="c", subcore_axis_name="s")

    @functools.partial(
        pl.kernel,
        mesh=mesh,
        compiler_params=_params,
        out_type=(
            jax.ShapeDtypeStruct((_DICT, D), jnp.float32),
            jax.ShapeDtypeStruct((2 * _NW, D), jnp.float32),
            jax.ShapeDtypeStruct((2 * _NW, _NL), jnp.int32),
        ),
        scratch_types=[
            pltpu.VMEM((2 * _BLK,), jnp.int32),
            pltpu.VMEM((2 * _BLK,), jnp.int32),
            pltpu.VMEM((2, _G, D), jnp.float32),
            pltpu.VMEM((4, D), jnp.float32),
            pltpu.VMEM((D,), jnp.float32),
            pltpu.VMEM((_MW,), jnp.float32),
            pltpu.VMEM((_NL,), jnp.int32),
            pltpu.SemaphoreType.DMA((2,)),
            pltpu.SemaphoreType.DMA((4,)),
        ],
    )
    def build(base_hbm, eidx_hbm, sidx_hbm, msk_hbm,
              table_hbm, psum_hbm, pmeta_hbm,
              sidx_v, eidx_v, rows_v, stage_v, acc_v, ones_v, meta_v,
              sem_g, sem_s):
        wid = lax.axis_index("s") * 2 + lax.axis_index("c")
        c0 = pl.multiple_of(wid * chunk, _G)
        c1 = jnp.minimum(c0 + chunk, T)
        lanes = lax.iota(jnp.int32, _NL)

        for k in range(_MW // _NL):
            ones_v[pl.ds(k * _NL, _NL)] = jnp.ones((_NL,), jnp.float32)

        a0 = pl.multiple_of((c0 // _BLK) * _BLK, _BLK)
        bs0 = (c0 // _BLK) % 2
        boffs0 = pl.multiple_of(bs0 * _BLK, 8)
        pltpu.sync_copy(sidx_hbm.at[pl.ds(a0, _BLK)],
                        sidx_v.at[pl.ds(boffs0, _BLK)])
        pltpu.sync_copy(eidx_hbm.at[pl.ds(a0, _BLK)],
                        eidx_v.at[pl.ds(boffs0, _BLK)])
        cur0 = _sload(sidx_v, c0 - a0 + bs0 * _BLK)

        bo0 = pl.multiple_of(c0 - a0 + bs0 * _BLK, _G)
        pg0 = (c0 // _G) % 2
        pltpu.async_copy(base_hbm.at[eidx_v.at[pl.ds(bo0, _G)]],
                         rows_v.at[pg0], sem_g.at[pg0])

        def write_partial(slot, seg, cnt):
            pltpu.sync_copy(acc_v, psum_hbm.at[slot])
            meta = jnp.where(lanes == 0, seg,
                             jnp.where(lanes == 1, cnt, jnp.int32(0)))
            meta_v[pl.ds(0, _NL)] = meta
            pltpu.sync_copy(meta_v, pmeta_hbm.at[slot])

        def body(j, st):
            cur, cnt, first, ncl = st
            i = c0 + j
            live = i < c1
            ig = i // _G
            pg = ig % 2

            @pl.when((i % _G == 0) & live)
            def _():
                inx = i + _G
                @pl.when((inx % _BLK == 0) & (inx < c1))
                def _():
                    ian = pl.multiple_of(inx, _BLK)
                    bsn = pl.multiple_of(((inx // _BLK) % 2) * _BLK, 8)
                    pltpu.sync_copy(sidx_hbm.at[pl.ds(ian, _BLK)],
                                    sidx_v.at[pl.ds(bsn, _BLK)])
                    pltpu.sync_copy(eidx_hbm.at[pl.ds(ian, _BLK)],
                                    eidx_v.at[pl.ds(bsn, _BLK)])

                pltpu.make_async_copy(base_hbm.at[pl.ds(0, _G)],
                                      rows_v.at[pg], sem_g.at[pg]).wait()

                @pl.when(inx < c1)
                def _():
                    bon = pl.multiple_of(
                        inx - (inx // _BLK) * _BLK
                        + ((inx // _BLK) % 2) * _BLK, _G)
                    pltpu.async_copy(
                        base_hbm.at[eidx_v.at[pl.ds(bon, _G)]],
                        rows_v.at[1 - pg], sem_g.at[1 - pg])

            boff = i - (i // _BLK) * _BLK + ((i // _BLK) % 2) * _BLK
            sv = _sload(sidx_v, boff)
            is_b = live & (sv != cur)
            close_int = is_b & (first == 0)
            close_first = is_b & (first == 1)

            @pl.when(close_int)
            def _():
                slot = ncl % 4

                @pl.when(ncl >= 4)
                def _():
                    pltpu.make_async_copy(
                        table_hbm.at[0], stage_v.at[slot],
                        sem_s.at[slot]).wait()
                    pltpu.make_async_copy(
                        msk_hbm.at[0], ones_v, sem_s.at[slot]).wait()

                den = jnp.full((_NL,), cnt.astype(jnp.float32))
                inv = jnp.ones((_NL,), jnp.float32) / den
                for k in range(nvec):
                    sl = pl.ds(k * _NL, _NL)
                    stage_v[slot, sl] = acc_v[sl] * inv
                pltpu.async_copy(stage_v.at[slot], table_hbm.at[cur],
                                 sem_s.at[slot])
                pltpu.async_copy(ones_v, msk_hbm.at[cur], sem_s.at[slot])

            @pl.when(close_first)
            def _():
                write_partial(2 * wid, cur, cnt)

            goff = i - ig * _G

            @pl.when(is_b)
            def _():
                for k in range(nvec):
                    sl = pl.ds(k * _NL, _NL)
                    acc_v[sl] = rows_v[pg, goff, sl]

            @pl.when(live & jnp.logical_not(is_b))
            def _():
                for k in range(nvec):
                    sl = pl.ds(k * _NL, _NL)
                    plsc.addupdate(acc_v.at[sl], rows_v[pg, goff, sl])

            cnt_new = jnp.where(is_b, jnp.int32(1),
                                jnp.where(live, cnt + 1, cnt))
            cur_new = jnp.where(is_b, sv, cur)
            ncl_new = jnp.where(close_int, ncl + 1, ncl)
            first_new = jnp.where(is_b, jnp.int32(0), first)
            return (cur_new, cnt_new, first_new, ncl_new)

        cur_f, cnt_f, first_f, ncl_f = lax.fori_loop(
            0, chunk, body, (cur0, jnp.int32(0), jnp.int32(1), jnp.int32(0)))

        @pl.when(first_f == 1)
        def _():
            write_partial(2 * wid, cur_f, cnt_f)
            meta_v[pl.ds(0, _NL)] = jnp.where(
                lanes == 0, jnp.int32(-1), jnp.int32(0))
            pltpu.sync_copy(meta_v, pmeta_hbm.at[2 * wid + 1])

        @pl.when(first_f == 0)
        def _():
            write_partial(2 * wid + 1, cur_f, cnt_f)

        for k in range(4):
            @pl.when(ncl_f > k)
            def _(k=k):
                pltpu.make_async_copy(
                    table_hbm.at[0], stage_v.at[k], sem_s.at[k]).wait()
                pltpu.make_async_copy(
                    msk_hbm.at[0], ones_v, sem_s.at[k]).wait()

    return build(base_weight, extract_index, scatter_index, msk0)


def _combine_gather(table, msk, psum, pmeta, flat_tokens):
    N = flat_tokens.shape[0]
    D = table.shape[1]
    nvec = D // _NL
    per_w = N // _NW
    blk = 64
    nparts = pmeta.shape[0]
    mesh = plsc.VectorSubcoreMesh(core_axis_name="c", subcore_axis_name="s")

    @functools.partial(
        pl.kernel,
        mesh=mesh,
        compiler_params=_params,
        out_type=jax.ShapeDtypeStruct((N, D), jnp.float32),
        scratch_types=[
            pltpu.VMEM((2 * _NW, D), jnp.float32),
            pltpu.VMEM((2 * _NW, _NL), jnp.int32),
            pltpu.VMEM((D,), jnp.float32),
            pltpu.VMEM((D,), jnp.float32),
            pltpu.VMEM((_MW,), jnp.float32),
            pltpu.VMEM((64,), jnp.int32),
            pltpu.VMEM((64, D), jnp.float32),
            pltpu.VMEM((64, _MW), jnp.float32),
            pltpu.SemaphoreType.DMA,
            pltpu.SemaphoreType.DMA,
        ],
    )
    def gat(table_hbm, msk_hbm, psum_hbm, pmeta_hbm, tok_hbm, out_hbm,
            parts_v, pmeta_v, sacc_v, sstage_v, ones_v, idx_v, rows_v,
            mrows_v, sem, sem2):
        sid = lax.axis_index("s")
        wid = sid * 2 + lax.axis_index("c")
        zv = jnp.zeros((_NL,), jnp.float32)

        @pl.when(sid == 0)
        def _():
            for k in range(_MW // _NL):
                ones_v[pl.ds(k * _NL, _NL)] = jnp.ones((_NL,), jnp.float32)
            pltpu.sync_copy(psum_hbm, parts_v)
            pltpu.sync_copy(pmeta_hbm, pmeta_v)

            def close(seg, cnt):
                den = jnp.full((_NL,), cnt.astype(jnp.float32))
                inv = jnp.ones((_NL,), jnp.float32) / den
                for k in range(nvec):
                    sl = pl.ds(k * _NL, _NL)
                    sstage_v[sl] = sacc_v[sl] * inv
                pltpu.sync_copy(sstage_v, table_hbm.at[seg])
                pltpu.sync_copy(ones_v, msk_hbm.at[seg])

            def body(e, st):
                cur, cnt = st
                id_e = _sload(pmeta_v, e, 0)
                cnt_e = _sload(pmeta_v, e, 1)
                valid = cnt_e > 0
                same = valid & (id_e == cur)
                newseg = valid & jnp.logical_not(same)
                @pl.when(newseg & (cur >= 0))
                def _():
                    close(cur, cnt)

                @pl.when(newseg)
                def _():
                    for k in range(nvec):
                        sacc_v[pl.ds(k * _NL, _NL)] = _row_vload(
                            parts_v, e, k * _NL)

                @pl.when(same)
                def _():
                    for k in range(nvec):
                        sl = pl.ds(k * _NL, _NL)
                        sacc_v[sl] = sacc_v[sl] + _row_vload(
                            parts_v, e, k * _NL)

                cur_new = jnp.where(newseg, id_e, cur)
                cnt_new = jnp.where(newseg, cnt_e,
                                    jnp.where(same, cnt + cnt_e, cnt))
                return (cur_new, cnt_new)

            cur_f, cnt_f = lax.fori_loop(
                0, nparts, body, (jnp.int32(-1), jnp.int32(0)))

            @pl.when(cur_f >= 0)
            def _():
                close(cur_f, cnt_f)

        plsc.subcore_barrier()

        base = wid * per_w

        def gbody(b, carry):
            off = pl.multiple_of(base + b * blk, blk)
            pltpu.sync_copy(tok_hbm.at[pl.ds(off, blk)], idx_v)
            cp1 = pltpu.async_copy(table_hbm.at[idx_v], rows_v, sem)
            cp2 = pltpu.async_copy(msk_hbm.at[idx_v], mrows_v, sem2)
            cp1.wait()
            cp2.wait()

            def mbody(j, mcarry):
                m = _sload(mrows_v, j, 0)

                @pl.when(m == 0.0)
                def _():
                    for k in range(nvec):
                        _row_vstore(rows_v, j, k * _NL, zv)
                return mcarry

            lax.fori_loop(0, blk, mbody, jnp.int32(0))
            pltpu.sync_copy(rows_v, out_hbm.at[pl.ds(off, blk)])
            return carry

        lax.fori_loop(0, per_w // blk, gbody, jnp.int32(0))

    return gat(table, msk, psum, pmeta, flat_tokens)


def kernel(base_weight, extract_index, scatter_index, token_index):
    B, L = token_index.shape
    D = base_weight.shape[1]
    z0 = base_weight[0, 0] * 0.0
    msk0 = jnp.full((_DICT, _MW), 0.0, jnp.float32) + z0
    table, psum, pmeta = _build_table(
        base_weight, extract_index, scatter_index, msk0)
    out = _combine_gather(table, msk0, psum, pmeta, token_index.reshape(-1))
    return out.reshape(B, L, D)


# speedup vs baseline: 5.9354x; 1.8967x over previous
import functools

import jax
import jax.numpy as jnp
from jax import lax
from jax.experimental import pallas as pl
from jax.experimental.pallas import tpu as pltpu
from jax.experimental.pallas import tpu_sc as plsc

_DICT = 100000
_NW = 32
_BLK = 480
_G = 16
_NL = 16
_MW = 128

_params = pltpu.CompilerParams(needs_layout_passes=False)


def _sload(ref, *idx):
    v = plsc.load_gather(ref, [jnp.full((_NL,), i, jnp.int32) for i in idx])
    return v[0]


def _row_vload(ref, row, col0):
    lanes = lax.iota(jnp.int32, _NL)
    return plsc.load_gather(
        ref, [jnp.full((_NL,), row, jnp.int32), col0 + lanes])


def _row_vstore(ref, row, col0, val):
    lanes = lax.iota(jnp.int32, _NL)
    plsc.store_scatter(
        ref, [jnp.full((_NL,), row, jnp.int32), col0 + lanes], val)


def _build_table(base_weight, extract_index, scatter_index, msk0):
    V, D = base_weight.shape
    T = extract_index.shape[0]
    nvec = D // _NL
    chunk = ((T // _NW) + _G - 1) // _G * _G
    ngrp = chunk // _G
    mesh = plsc.VectorSubcoreMesh(core_axis_name="c", subcore_axis_name="s")

    @functools.partial(
        pl.kernel,
        mesh=mesh,
        compiler_params=_params,
        out_type=(
            jax.ShapeDtypeStruct((_DICT, D), jnp.float32),
            jax.ShapeDtypeStruct((2 * _NW, D), jnp.float32),
            jax.ShapeDtypeStruct((2 * _NW, _NL), jnp.int32),
        ),
        scratch_types=[
            pltpu.VMEM((2 * _BLK,), jnp.int32),
            pltpu.VMEM((2 * _BLK,), jnp.int32),
            pltpu.VMEM((2, _G, D), jnp.float32),
            pltpu.VMEM((4, D), jnp.float32),
            pltpu.VMEM((D,), jnp.float32),
            pltpu.VMEM((_MW,), jnp.float32),
            pltpu.VMEM((_NL,), jnp.int32),
            pltpu.SemaphoreType.DMA((2,)),
            pltpu.SemaphoreType.DMA((4,)),
        ],
    )
    def build(base_hbm, eidx_hbm, sidx_hbm, msk_hbm,
              table_hbm, psum_hbm, pmeta_hbm,
              sidx_v, eidx_v, rows_v, stage_v, acc_v, ones_v, meta_v,
              sem_g, sem_s):
        wid = lax.axis_index("s") * 2 + lax.axis_index("c")
        c0 = pl.multiple_of(wid * chunk, _G)
        c1 = jnp.minimum(c0 + chunk, T)
        lanes = lax.iota(jnp.int32, _NL)

        for k in range(_MW // _NL):
            ones_v[pl.ds(k * _NL, _NL)] = jnp.ones((_NL,), jnp.float32)

        a0 = pl.multiple_of((c0 // _BLK) * _BLK, _BLK)
        bs0 = (c0 // _BLK) % 2
        boffs0 = pl.multiple_of(bs0 * _BLK, 8)
        pltpu.sync_copy(sidx_hbm.at[pl.ds(a0, _BLK)],
                        sidx_v.at[pl.ds(boffs0, _BLK)])
        pltpu.sync_copy(eidx_hbm.at[pl.ds(a0, _BLK)],
                        eidx_v.at[pl.ds(boffs0, _BLK)])
        cur0 = _sload(sidx_v, c0 - a0 + bs0 * _BLK)

        bo0 = pl.multiple_of(c0 - a0 + bs0 * _BLK, _G)
        pg0 = (c0 // _G) % 2
        pltpu.async_copy(base_hbm.at[eidx_v.at[pl.ds(bo0, _G)]],
                         rows_v.at[pg0], sem_g.at[pg0])

        def write_partial(slot, seg, cnt):
            pltpu.sync_copy(acc_v, psum_hbm.at[slot])
            meta = jnp.where(lanes == 0, seg,
                             jnp.where(lanes == 1, cnt, jnp.int32(0)))
            meta_v[pl.ds(0, _NL)] = meta
            pltpu.sync_copy(meta_v, pmeta_hbm.at[slot])

        def body(j, st):
            cur, cnt, first, ncl = st
            i = c0 + j
            live = i < c1
            ig = i // _G
            pg = ig % 2

            @pl.when((i % _G == 0) & live)
            def _():
                inx = i + _G
                @pl.when((inx % _BLK == 0) & (inx < c1))
                def _():
                    ian = pl.multiple_of(inx, _BLK)
                    bsn = pl.multiple_of(((inx // _BLK) % 2) * _BLK, 8)
                    pltpu.sync_copy(sidx_hbm.at[pl.ds(ian, _BLK)],
                                    sidx_v.at[pl.ds(bsn, _BLK)])
                    pltpu.sync_copy(eidx_hbm.at[pl.ds(ian, _BLK)],
                                    eidx_v.at[pl.ds(bsn, _BLK)])

                pltpu.make_async_copy(base_hbm.at[pl.ds(0, _G)],
                                      rows_v.at[pg], sem_g.at[pg]).wait()

                @pl.when(inx < c1)
                def _():
                    bon = pl.multiple_of(
                        inx - (inx // _BLK) * _BLK
                        + ((inx // _BLK) % 2) * _BLK, _G)
                    pltpu.async_copy(
                        base_hbm.at[eidx_v.at[pl.ds(bon, _G)]],
                        rows_v.at[1 - pg], sem_g.at[1 - pg])

            boff = i - (i // _BLK) * _BLK + ((i // _BLK) % 2) * _BLK
            sv = _sload(sidx_v, boff)
            is_b = live & (sv != cur)
            close_int = is_b & (first == 0)
            close_first = is_b & (first == 1)

            @pl.when(close_int)
            def _():
                slot = ncl % 4

                @pl.when(ncl >= 4)
                def _():
                    pltpu.make_async_copy(
                        table_hbm.at[0], stage_v.at[slot],
                        sem_s.at[slot]).wait()
                    pltpu.make_async_copy(
                        msk_hbm.at[0], ones_v, sem_s.at[slot]).wait()

                den = jnp.full((_NL,), cnt.astype(jnp.float32))
                inv = jnp.ones((_NL,), jnp.float32) / den

                @plsc.parallel_loop(0, D, step=_NL, unroll=8)
                def _(o):
                    sl = pl.ds(pl.multiple_of(o, _NL), _NL)
                    stage_v[slot, sl] = acc_v[sl] * inv
                pltpu.async_copy(stage_v.at[slot], table_hbm.at[cur],
                                 sem_s.at[slot])
                pltpu.async_copy(ones_v, msk_hbm.at[cur], sem_s.at[slot])

            @pl.when(close_first)
            def _():
                write_partial(2 * wid, cur, cnt)

            goff = i - ig * _G

            @pl.when(live)
            def _():
                @plsc.parallel_loop(0, D, step=_NL, unroll=8)
                def _(o):
                    sl = pl.ds(pl.multiple_of(o, _NL), _NL)
                    row_k = rows_v[pg, goff, sl]
                    acc_v[sl] = jnp.where(is_b, row_k, acc_v[sl] + row_k)

            cnt_new = jnp.where(is_b, jnp.int32(1),
                                jnp.where(live, cnt + 1, cnt))
            cur_new = jnp.where(is_b, sv, cur)
            ncl_new = jnp.where(close_int, ncl + 1, ncl)
            first_new = jnp.where(is_b, jnp.int32(0), first)
            return (cur_new, cnt_new, first_new, ncl_new)

        cur_f, cnt_f, first_f, ncl_f = lax.fori_loop(
            0, chunk, body, (cur0, jnp.int32(0), jnp.int32(1), jnp.int32(0)))

        @pl.when(first_f == 1)
        def _():
            write_partial(2 * wid, cur_f, cnt_f)
            meta_v[pl.ds(0, _NL)] = jnp.where(
                lanes == 0, jnp.int32(-1), jnp.int32(0))
            pltpu.sync_copy(meta_v, pmeta_hbm.at[2 * wid + 1])

        @pl.when(first_f == 0)
        def _():
            write_partial(2 * wid + 1, cur_f, cnt_f)

        for k in range(4):
            @pl.when(ncl_f > k)
            def _(k=k):
                pltpu.make_async_copy(
                    table_hbm.at[0], stage_v.at[k], sem_s.at[k]).wait()
                pltpu.make_async_copy(
                    msk_hbm.at[0], ones_v, sem_s.at[k]).wait()

    return build(base_weight, extract_index, scatter_index, msk0)


def _combine_gather(table, msk, psum, pmeta, flat_tokens):
    N = flat_tokens.shape[0]
    D = table.shape[1]
    nvec = D // _NL
    per_w = N // _NW
    blk = 64
    nparts = pmeta.shape[0]
    mesh = plsc.VectorSubcoreMesh(core_axis_name="c", subcore_axis_name="s")

    @functools.partial(
        pl.kernel,
        mesh=mesh,
        compiler_params=_params,
        out_type=jax.ShapeDtypeStruct((N, D), jnp.float32),
        scratch_types=[
            pltpu.VMEM((2 * _NW, D), jnp.float32),
            pltpu.VMEM((2 * _NW, _NL), jnp.int32),
            pltpu.VMEM((D,), jnp.float32),
            pltpu.VMEM((D,), jnp.float32),
            pltpu.VMEM((_MW,), jnp.float32),
            pltpu.VMEM((64,), jnp.int32),
            pltpu.VMEM((64, D), jnp.float32),
            pltpu.VMEM((64, _MW), jnp.float32),
            pltpu.SemaphoreType.DMA,
            pltpu.SemaphoreType.DMA,
        ],
    )
    def gat(table_hbm, msk_hbm, psum_hbm, pmeta_hbm, tok_hbm, out_hbm,
            parts_v, pmeta_v, sacc_v, sstage_v, ones_v, idx_v, rows_v,
            mrows_v, sem, sem2):
        sid = lax.axis_index("s")
        wid = sid * 2 + lax.axis_index("c")
        zv = jnp.zeros((_NL,), jnp.float32)

        @pl.when(sid == 0)
        def _():
            for k in range(_MW // _NL):
                ones_v[pl.ds(k * _NL, _NL)] = jnp.ones((_NL,), jnp.float32)
            pltpu.sync_copy(psum_hbm, parts_v)
            pltpu.sync_copy(pmeta_hbm, pmeta_v)

            def close(seg, cnt):
                den = jnp.full((_NL,), cnt.astype(jnp.float32))
                inv = jnp.ones((_NL,), jnp.float32) / den
                for k in range(nvec):
                    sl = pl.ds(k * _NL, _NL)
                    sstage_v[sl] = sacc_v[sl] * inv
                pltpu.sync_copy(sstage_v, table_hbm.at[seg])
                pltpu.sync_copy(ones_v, msk_hbm.at[seg])

            def body(e, st):
                cur, cnt = st
                id_e = _sload(pmeta_v, e, 0)
                cnt_e = _sload(pmeta_v, e, 1)
                valid = cnt_e > 0
                same = valid & (id_e == cur)
                newseg = valid & jnp.logical_not(same)
                @pl.when(newseg & (cur >= 0))
                def _():
                    close(cur, cnt)

                @pl.when(newseg)
                def _():
                    for k in range(nvec):
                        sacc_v[pl.ds(k * _NL, _NL)] = _row_vload(
                            parts_v, e, k * _NL)

                @pl.when(same)
                def _():
                    for k in range(nvec):
                        sl = pl.ds(k * _NL, _NL)
                        sacc_v[sl] = sacc_v[sl] + _row_vload(
                            parts_v, e, k * _NL)

                cur_new = jnp.where(newseg, id_e, cur)
                cnt_new = jnp.where(newseg, cnt_e,
                                    jnp.where(same, cnt + cnt_e, cnt))
                return (cur_new, cnt_new)

            cur_f, cnt_f = lax.fori_loop(
                0, nparts, body, (jnp.int32(-1), jnp.int32(0)))

            @pl.when(cur_f >= 0)
            def _():
                close(cur_f, cnt_f)

        plsc.subcore_barrier()

        base = wid * per_w

        def gbody(b, carry):
            off = pl.multiple_of(base + b * blk, blk)
            pltpu.sync_copy(tok_hbm.at[pl.ds(off, blk)], idx_v)
            cp1 = pltpu.async_copy(table_hbm.at[idx_v], rows_v, sem)
            cp2 = pltpu.async_copy(msk_hbm.at[idx_v], mrows_v, sem2)
            cp1.wait()
            cp2.wait()

            def mbody(j, mcarry):
                m = _sload(mrows_v, j, 0)

                @pl.when(m == 0.0)
                def _():
                    for k in range(nvec):
                        _row_vstore(rows_v, j, k * _NL, zv)
                return mcarry

            lax.fori_loop(0, blk, mbody, jnp.int32(0))
            pltpu.sync_copy(rows_v, out_hbm.at[pl.ds(off, blk)])
            return carry

        lax.fori_loop(0, per_w // blk, gbody, jnp.int32(0))

    return gat(table, msk, psum, pmeta, flat_tokens)


def kernel(base_weight, extract_index, scatter_index, token_index):
    B, L = token_index.shape
    D = base_weight.shape[1]
    z0 = base_weight[0, 0] * 0.0
    msk0 = jnp.full((_DICT, _MW), 0.0, jnp.float32) + z0
    table, psum, pmeta = _build_table(
        base_weight, extract_index, scatter_index, msk0)
    out = _combine_gather(table, msk0, psum, pmeta, token_index.reshape(-1))
    return out.reshape(B, L, D)
